# baseline (device time: 15984 ns/iter reference)
import jax
import jax.numpy as jnp
from jax import lax
from jax.experimental import pallas as pl
from jax.experimental.pallas import tpu as pltpu

N_DEV = 4
NSUB = 4


def kernel(partial, resid, gamma):
    g = gamma.reshape(1, -1)
    _, m, n = partial.shape
    blk = m // N_DEV
    sub = blk // NSUB

    def body(p_ref, resid_hbm, g_ref, out_hbm,
             x_vmem, r_vmem, out_vmem, xb_ref, rs_buf, ag_buf,
             in_sems, out_sems, send_sems1, recv_sems1, send_sems2, recv_sems2):
        my = lax.axis_index("i")

        xcopy = pltpu.make_async_copy(p_ref.at[0], x_vmem, in_sems.at[0])
        xcopy.start()
        rcopy = pltpu.make_async_copy(
            resid_hbm.at[pl.ds(my * blk, blk), :], r_vmem, in_sems.at[1])
        rcopy.start()

        barrier = pltpu.get_barrier_semaphore()
        for k in range(1, N_DEV):
            p = (my + k) % N_DEV
            pl.semaphore_signal(barrier, inc=1, device_id=(p,),
                                device_id_type=pl.DeviceIdType.MESH)

        xcopy.wait()
        xb_ref[:, :] = x_vmem[:, :].astype(jnp.bfloat16)
        for s in range(NSUB):
            rs_buf[pl.ds(my, 1), s] = jnp.zeros((1, sub, n), jnp.bfloat16)

        pl.semaphore_wait(barrier, N_DEV - 1)

        rs_sends = []
        for s in range(NSUB):
            for k in (2, 1, 3):
                p = (my + k) % N_DEV
                rdma = pltpu.make_async_remote_copy(
                    src_ref=xb_ref.at[pl.ds(p * blk + s * sub, sub), :],
                    dst_ref=rs_buf.at[my, s],
                    send_sem=send_sems1.at[k - 1, s],
                    recv_sem=recv_sems1.at[my, s],
                    device_id=(p,),
                    device_id_type=pl.DeviceIdType.MESH,
                )
                rdma.start()
                rs_sends.append(rdma)

        rcopy.wait()
        out_copies = []

        ag_sends = []
        for s in range(NSUB):
            for k in range(1, N_DEV):
                q = (my + k) % N_DEV
                recv = pltpu.make_async_remote_copy(
                    src_ref=rs_buf.at[q, s],
                    dst_ref=rs_buf.at[q, s],
                    send_sem=send_sems1.at[k - 1, s],
                    recv_sem=recv_sems1.at[q, s],
                    device_id=(q,),
                    device_id_type=pl.DeviceIdType.MESH,
                )
                recv.wait_recv()

            rows = pl.ds(my * blk + s * sub, sub)
            y = (x_vmem[rows, :]
                 + (rs_buf[0, s] + rs_buf[1, s]
                    + rs_buf[2, s] + rs_buf[3, s]).astype(jnp.float32)
                 + r_vmem[pl.ds(s * sub, sub), :])
            rms = jnp.sqrt(jnp.mean(y * y, axis=-1, keepdims=True) + 1e-6)
            z = y / rms * g_ref[0, :]
            ag_buf[pl.ds(my, 1), s] = z.astype(jnp.bfloat16)[None]

            for k in (2, 1, 3):
                p = (my + k) % N_DEV
                rdma = pltpu.make_async_remote_copy(
                    src_ref=ag_buf.at[my, s],
                    dst_ref=ag_buf.at[my, s],
                    send_sem=send_sems2.at[k - 1, s],
                    recv_sem=recv_sems2.at[my, s],
                    device_id=(p,),
                    device_id_type=pl.DeviceIdType.MESH,
                )
                rdma.start()
                ag_sends.append(rdma)

            out_vmem[rows, :] = z
            cp = pltpu.make_async_copy(
                out_vmem.at[rows, :], out_hbm.at[rows, :],
                out_sems.at[N_DEV - 1, s])
            cp.start()
            out_copies.append(cp)

        for s in range(NSUB):
            for k in range(1, N_DEV):
                q = (my + k) % N_DEV
                recv = pltpu.make_async_remote_copy(
                    src_ref=ag_buf.at[q, s],
                    dst_ref=ag_buf.at[q, s],
                    send_sem=send_sems2.at[k - 1, s],
                    recv_sem=recv_sems2.at[q, s],
                    device_id=(q,),
                    device_id_type=pl.DeviceIdType.MESH,
                )
                recv.wait_recv()
                rows_q = pl.ds(q * blk + s * sub, sub)
                out_vmem[rows_q, :] = ag_buf[pl.ds(q, 1), s].astype(jnp.float32)[0]
                cp = pltpu.make_async_copy(
                    out_vmem.at[rows_q, :], out_hbm.at[rows_q, :],
                    out_sems.at[k - 1, s])
                cp.start()
                out_copies.append(cp)

        for cp in out_copies:
            cp.wait()
        for rdma in rs_sends + ag_sends:
            rdma.wait_send()

    return pl.pallas_call(
        body,
        out_shape=jax.ShapeDtypeStruct((m, n), jnp.float32),
        in_specs=[
            pl.BlockSpec(memory_space=pltpu.MemorySpace.HBM),
            pl.BlockSpec(memory_space=pltpu.MemorySpace.HBM),
            pl.BlockSpec(memory_space=pltpu.VMEM),
        ],
        out_specs=pl.BlockSpec(memory_space=pltpu.MemorySpace.HBM),
        scratch_shapes=[
            pltpu.VMEM((m, n), jnp.float32),
            pltpu.VMEM((blk, n), jnp.float32),
            pltpu.VMEM((m, n), jnp.float32),
            pltpu.VMEM((m, n), jnp.bfloat16),
            pltpu.VMEM((N_DEV, NSUB, sub, n), jnp.bfloat16),
            pltpu.VMEM((N_DEV, NSUB, sub, n), jnp.bfloat16),
            pltpu.SemaphoreType.DMA((2,)),
            pltpu.SemaphoreType.DMA((N_DEV, NSUB)),
            pltpu.SemaphoreType.DMA((N_DEV - 1, NSUB)),
            pltpu.SemaphoreType.DMA((N_DEV, NSUB)),
            pltpu.SemaphoreType.DMA((N_DEV - 1, NSUB)),
            pltpu.SemaphoreType.DMA((N_DEV, NSUB)),
        ],
        compiler_params=pltpu.CompilerParams(collective_id=0),
    )(partial, resid, g)


# device time: 15565 ns/iter; 1.0269x vs baseline; 1.0269x over previous
import jax
import jax.numpy as jnp
from jax import lax
from jax.experimental import pallas as pl
from jax.experimental.pallas import tpu as pltpu

N_DEV = 4
NSUB = 4


def kernel(partial, resid, gamma):
    x = partial[0]
    g = gamma.reshape(1, -1)
    m, n = x.shape
    blk = m // N_DEV
    sub = blk // NSUB

    def body(x_ref, resid_ref, g_ref, out_ref,
             xb_ref, rs_buf, ag_buf,
             send_sems1, recv_sems1, send_sems2, recv_sems2):
        my = lax.axis_index("i")

        barrier = pltpu.get_barrier_semaphore()
        for k in range(1, N_DEV):
            p = (my + k) % N_DEV
            pl.semaphore_signal(barrier, inc=1, device_id=(p,),
                                device_id_type=pl.DeviceIdType.MESH)

        xb_ref[:, :] = x_ref[:, :].astype(jnp.bfloat16)
        for s in range(NSUB):
            rs_buf[pl.ds(my, 1), s] = jnp.zeros((1, sub, n), jnp.bfloat16)

        pl.semaphore_wait(barrier, N_DEV - 1)

        rs_sends = []
        for s in range(NSUB):
            for k in (2, 1, 3):
                p = (my + k) % N_DEV
                rdma = pltpu.make_async_remote_copy(
                    src_ref=xb_ref.at[pl.ds(p * blk + s * sub, sub), :],
                    dst_ref=rs_buf.at[my, s],
                    send_sem=send_sems1.at[k - 1, s],
                    recv_sem=recv_sems1.at[my, s],
                    device_id=(p,),
                    device_id_type=pl.DeviceIdType.MESH,
                )
                rdma.start()
                rs_sends.append(rdma)

        ag_sends = []
        for s in range(NSUB):
            for k in range(1, N_DEV):
                q = (my + k) % N_DEV
                recv = pltpu.make_async_remote_copy(
                    src_ref=rs_buf.at[q, s],
                    dst_ref=rs_buf.at[q, s],
                    send_sem=send_sems1.at[k - 1, s],
                    recv_sem=recv_sems1.at[q, s],
                    device_id=(q,),
                    device_id_type=pl.DeviceIdType.MESH,
                )
                recv.wait_recv()

            rows = pl.ds(my * blk + s * sub, sub)
            y = (x_ref[rows, :]
                 + (rs_buf[0, s] + rs_buf[1, s]
                    + rs_buf[2, s] + rs_buf[3, s]).astype(jnp.float32)
                 + resid_ref[rows, :])
            rms = jnp.sqrt(jnp.mean(y * y, axis=-1, keepdims=True) + 1e-6)
            z = y / rms * g_ref[0, :]
            ag_buf[pl.ds(my, 1), s] = z.astype(jnp.bfloat16)[None]

            for k in (2, 1, 3):
                p = (my + k) % N_DEV
                rdma = pltpu.make_async_remote_copy(
                    src_ref=ag_buf.at[my, s],
                    dst_ref=ag_buf.at[my, s],
                    send_sem=send_sems2.at[k - 1, s],
                    recv_sem=recv_sems2.at[my, s],
                    device_id=(p,),
                    device_id_type=pl.DeviceIdType.MESH,
                )
                rdma.start()
                ag_sends.append(rdma)

            out_ref[rows, :] = z

        for s in range(NSUB):
            for k in range(1, N_DEV):
                q = (my + k) % N_DEV
                recv = pltpu.make_async_remote_copy(
                    src_ref=ag_buf.at[q, s],
                    dst_ref=ag_buf.at[q, s],
                    send_sem=send_sems2.at[k - 1, s],
                    recv_sem=recv_sems2.at[q, s],
                    device_id=(q,),
                    device_id_type=pl.DeviceIdType.MESH,
                )
                recv.wait_recv()
                out_ref[pl.ds(q * blk + s * sub, sub), :] = (
                    ag_buf[pl.ds(q, 1), s].astype(jnp.float32)[0])

        for rdma in rs_sends + ag_sends:
            rdma.wait_send()

    return pl.pallas_call(
        body,
        out_shape=jax.ShapeDtypeStruct((m, n), jnp.float32),
        in_specs=[
            pl.BlockSpec(memory_space=pltpu.VMEM),
            pl.BlockSpec(memory_space=pltpu.VMEM),
            pl.BlockSpec(memory_space=pltpu.VMEM),
        ],
        out_specs=pl.BlockSpec(memory_space=pltpu.VMEM),
        scratch_shapes=[
            pltpu.VMEM((m, n), jnp.bfloat16),
            pltpu.VMEM((N_DEV, NSUB, sub, n), jnp.bfloat16),
            pltpu.VMEM((N_DEV, NSUB, sub, n), jnp.bfloat16),
            pltpu.SemaphoreType.DMA((N_DEV - 1, NSUB)),
            pltpu.SemaphoreType.DMA((N_DEV, NSUB)),
            pltpu.SemaphoreType.DMA((N_DEV - 1, NSUB)),
            pltpu.SemaphoreType.DMA((N_DEV, NSUB)),
        ],
        compiler_params=pltpu.CompilerParams(collective_id=0),
    )(x, resid, g)
